# baseline (speedup 1.0000x reference)
import functools

import jax
import jax.numpy as jnp
from jax import lax
from jax.experimental import pallas as pl
from jax.experimental.pallas import tpu as pltpu
from jax.experimental.pallas import tpu_sc as plsc

_NC = 2
_NS = 16
_NW = _NC * _NS


@functools.partial(jax.jit, static_argnums=(2, 3, 4, 5, 6))
def _emb_gather(x_flat, table, B, D, C, NBUF, K):
    b_per_w = B // _NW
    n_chunks = b_per_w // C
    assert n_chunks * C == b_per_w
    assert n_chunks % NBUF == 0 and NBUF > K >= 1
    mesh = plsc.VectorSubcoreMesh(core_axis_name="c", subcore_axis_name="s")

    @functools.partial(
        pl.kernel,
        out_type=jax.ShapeDtypeStruct((B, D), jnp.int32),
        mesh=mesh,
        scratch_types=[
            pltpu.VMEM((NBUF, C), jnp.int32),
            pltpu.VMEM((NBUF, C, D), jnp.int32),
            pltpu.SemaphoreType.DMA((NBUF,)),
            pltpu.SemaphoreType.DMA((NBUF,)),
        ],
        compiler_params=pltpu.CompilerParams(use_tc_tiling_on_sc=False),
    )
    def k(x_hbm, table_hbm, out_hbm, idx_v, rows_v, s_g, s_st):
        wid = lax.axis_index("s") * _NC + lax.axis_index("c")
        base = wid * b_per_w

        def gather_copy(b):
            return pltpu.make_async_copy(
                table_hbm.at[idx_v.at[b]], rows_v.at[b], s_g.at[b])

        def store_copy(g, b):
            return pltpu.make_async_copy(
                rows_v.at[b], out_hbm.at[pl.ds(base + g * C, C)], s_st.at[b])

        def outer(g2, carry):
            for j in range(NBUF):
                g = g2 * NBUF + j

                @pl.when(g >= NBUF)
                def _():
                    store_copy(g - NBUF, j).wait()

                pltpu.sync_copy(x_hbm.at[pl.ds(base + g * C, C)],
                                idx_v.at[j])
                gather_copy(j).start()

                jd = (j - K) % NBUF

                @pl.when(g >= K)
                def _():
                    gather_copy(jd).wait()
                    store_copy(g - K, jd).start()

            return carry

        lax.fori_loop(0, n_chunks // NBUF, outer, 0)

        for c in range(n_chunks - K, n_chunks):
            b = c % NBUF
            gather_copy(b).wait()
            store_copy(c, b).start()
        for c in range(n_chunks - NBUF, n_chunks):
            store_copy(c, c % NBUF).wait()

    return k(x_flat, table)


def kernel(x, table):
    B = x.shape[0] * x.shape[1]
    D = table.shape[1]
    table_c = lax.bitcast_convert_type(
        table.astype(jnp.bfloat16).reshape(table.shape[0], D // 2, 2),
        jnp.int32)
    out = _emb_gather(x.reshape(B).astype(jnp.int32), table_c, B, D // 2,
                      2048, 2, 1)
    out = lax.bitcast_convert_type(
        out.reshape(B, D // 2, 1), jnp.bfloat16).reshape(B, D)
    return out.astype(jnp.float32).reshape(x.shape[0], x.shape[1], D)


# speedup vs baseline: 1.5790x; 1.5790x over previous
import functools

import jax
import jax.numpy as jnp
from jax import lax
from jax.experimental import pallas as pl
from jax.experimental.pallas import tpu as pltpu
from jax.experimental.pallas import tpu_sc as plsc

_NC = 2
_NS = 16
_NW = _NC * _NS


def _downcast_table(table):
    R = 8000
    n, d = table.shape

    def body(t_ref, o_ref):
        o_ref[...] = t_ref[...].astype(jnp.bfloat16)

    return pl.pallas_call(
        body,
        grid=(n // R,),
        in_specs=[pl.BlockSpec((R, d), lambda i: (i, 0))],
        out_specs=pl.BlockSpec((R, d), lambda i: (i, 0)),
        out_shape=jax.ShapeDtypeStruct((n, d), jnp.bfloat16),
    )(table)


def _upcast_out(out_bf):
    R = 16384
    n, d = out_bf.shape

    def body(t_ref, o_ref):
        o_ref[...] = t_ref[...].astype(jnp.float32)

    return pl.pallas_call(
        body,
        grid=(n // R,),
        in_specs=[pl.BlockSpec((R, d), lambda i: (i, 0))],
        out_specs=pl.BlockSpec((R, d), lambda i: (i, 0)),
        out_shape=jax.ShapeDtypeStruct((n, d), jnp.float32),
    )(out_bf)


@functools.partial(jax.jit, static_argnums=(2, 3, 4, 5, 6))
def _emb_gather(x_flat, table, B, D, C, NBUF, K):
    b_per_w = B // _NW
    n_chunks = b_per_w // C
    assert n_chunks * C == b_per_w
    assert n_chunks % NBUF == 0 and NBUF > K >= 1
    mesh = plsc.VectorSubcoreMesh(core_axis_name="c", subcore_axis_name="s")

    @functools.partial(
        pl.kernel,
        out_type=jax.ShapeDtypeStruct((B, D), jnp.bfloat16),
        mesh=mesh,
        scratch_types=[
            pltpu.VMEM((NBUF, C), jnp.int32),
            pltpu.VMEM((NBUF, C, D), jnp.bfloat16),
            pltpu.SemaphoreType.DMA((NBUF,)),
            pltpu.SemaphoreType.DMA((NBUF,)),
        ],
        compiler_params=pltpu.CompilerParams(use_tc_tiling_on_sc=False),
    )
    def k(x_hbm, table_hbm, out_hbm, idx_v, rows_v, s_g, s_st):
        wid = lax.axis_index("s") * _NC + lax.axis_index("c")
        base = wid * b_per_w

        def gather_copy(b):
            return pltpu.make_async_copy(
                table_hbm.at[idx_v.at[b]], rows_v.at[b], s_g.at[b])

        def store_copy(g, b):
            return pltpu.make_async_copy(
                rows_v.at[b], out_hbm.at[pl.ds(base + g * C, C)], s_st.at[b])

        def outer(g2, carry):
            for j in range(NBUF):
                g = g2 * NBUF + j

                @pl.when(g >= NBUF)
                def _():
                    store_copy(g - NBUF, j).wait()

                pltpu.sync_copy(x_hbm.at[pl.ds(base + g * C, C)],
                                idx_v.at[j])
                gather_copy(j).start()

                jd = (j - K) % NBUF

                @pl.when(g >= K)
                def _():
                    gather_copy(jd).wait()
                    store_copy(g - K, jd).start()

            return carry

        lax.fori_loop(0, n_chunks // NBUF, outer, 0)

        for c in range(n_chunks - K, n_chunks):
            b = c % NBUF
            gather_copy(b).wait()
            store_copy(c, b).start()
        for c in range(n_chunks - NBUF, n_chunks):
            store_copy(c, c % NBUF).wait()

    return k(x_flat, table)


def kernel(x, table):
    B = x.shape[0] * x.shape[1]
    D = table.shape[1]
    table_bf = _downcast_table(table)
    out_bf = _emb_gather(x.reshape(B).astype(jnp.int32), table_bf, B, D,
                         2048, 2, 1)
    return _upcast_out(out_bf).reshape(x.shape[0], x.shape[1], D)


# speedup vs baseline: 1.6320x; 1.0335x over previous
import functools

import jax
import jax.numpy as jnp
from jax import lax
from jax.experimental import pallas as pl
from jax.experimental.pallas import tpu as pltpu
from jax.experimental.pallas import tpu_sc as plsc

_NC = 2
_NS = 16
_NW = _NC * _NS


def _pack_table(table):
    n, d = table.shape
    R = 8000
    h = d // 2

    def body(t_ref, o_ref):
        u = lax.bitcast_convert_type(t_ref[...], jnp.uint32)
        r = (u + jnp.uint32(0x7FFF) + ((u >> 16) & jnp.uint32(1))) >> 16
        w = (r[:, h:] << 16) | r[:, :h]
        o_ref[...] = lax.bitcast_convert_type(w, jnp.int32)

    return pl.pallas_call(
        body,
        grid=(n // R,),
        in_specs=[pl.BlockSpec((R, d), lambda i: (i, 0))],
        out_specs=pl.BlockSpec((R, h), lambda i: (i, 0)),
        out_shape=jax.ShapeDtypeStruct((n, h), jnp.int32),
    )(table)


def _unpack_out(packed, d):
    n, h = packed.shape
    R = 16384

    def body(p_ref, o_ref):
        u = lax.bitcast_convert_type(p_ref[...], jnp.uint32)
        lo = lax.bitcast_convert_type(u << 16, jnp.float32)
        hi = lax.bitcast_convert_type(u & jnp.uint32(0xFFFF0000), jnp.float32)
        o_ref[...] = jnp.concatenate([lo, hi], axis=1)

    return pl.pallas_call(
        body,
        grid=(n // R,),
        in_specs=[pl.BlockSpec((R, h), lambda i: (i, 0))],
        out_specs=pl.BlockSpec((R, d), lambda i: (i, 0)),
        out_shape=jax.ShapeDtypeStruct((n, d), jnp.float32),
    )(packed)


@functools.partial(jax.jit, static_argnums=(2, 3, 4, 5, 6))
def _emb_gather(x_flat, table, B, D, C, NBUF, K):
    b_per_w = B // _NW
    n_chunks = b_per_w // C
    assert n_chunks * C == b_per_w
    assert n_chunks % NBUF == 0 and NBUF > K >= 1
    mesh = plsc.VectorSubcoreMesh(core_axis_name="c", subcore_axis_name="s")

    @functools.partial(
        pl.kernel,
        out_type=jax.ShapeDtypeStruct((B, D), jnp.int32),
        mesh=mesh,
        scratch_types=[
            pltpu.VMEM((NBUF, C), jnp.int32),
            pltpu.VMEM((NBUF, C, D), jnp.int32),
            pltpu.SemaphoreType.DMA((NBUF,)),
            pltpu.SemaphoreType.DMA((NBUF,)),
        ],
        compiler_params=pltpu.CompilerParams(use_tc_tiling_on_sc=False),
    )
    def k(x_hbm, table_hbm, out_hbm, idx_v, rows_v, s_g, s_st):
        wid = lax.axis_index("s") * _NC + lax.axis_index("c")
        base = wid * b_per_w

        def gather_copy(b):
            return pltpu.make_async_copy(
                table_hbm.at[idx_v.at[b]], rows_v.at[b], s_g.at[b])

        def store_copy(g, b):
            return pltpu.make_async_copy(
                rows_v.at[b], out_hbm.at[pl.ds(base + g * C, C)], s_st.at[b])

        def outer(g2, carry):
            for j in range(NBUF):
                g = g2 * NBUF + j

                @pl.when(g >= NBUF)
                def _():
                    store_copy(g - NBUF, j).wait()

                pltpu.sync_copy(x_hbm.at[pl.ds(base + g * C, C)],
                                idx_v.at[j])
                gather_copy(j).start()

                jd = (j - K) % NBUF

                @pl.when(g >= K)
                def _():
                    gather_copy(jd).wait()
                    store_copy(g - K, jd).start()

            return carry

        lax.fori_loop(0, n_chunks // NBUF, outer, 0)

        for c in range(n_chunks - K, n_chunks):
            b = c % NBUF
            gather_copy(b).wait()
            store_copy(c, b).start()
        for c in range(n_chunks - NBUF, n_chunks):
            store_copy(c, c % NBUF).wait()

    return k(x_flat, table)


def kernel(x, table):
    B = x.shape[0] * x.shape[1]
    D = table.shape[1]
    table_p = _pack_table(table)
    out_p = _emb_gather(x.reshape(B).astype(jnp.int32), table_p, B, D // 2,
                        2048, 2, 1)
    return _unpack_out(out_p, D).reshape(x.shape[0], x.shape[1], D)


# speedup vs baseline: 2.4915x; 1.5267x over previous
import functools

import jax
import jax.numpy as jnp
from jax import lax
from jax.experimental import pallas as pl
from jax.experimental.pallas import tpu as pltpu
from jax.experimental.pallas import tpu_sc as plsc

_NC = 2
_NS = 16
_NW = _NC * _NS
_L = 16


@functools.partial(jax.jit, static_argnums=(2, 3, 4, 5))
def _emb_gather(x_flat, table2, B, C, NBUF, K):
    b_per_w = B // _NW
    n_chunks = b_per_w // C
    assert n_chunks * C == b_per_w
    assert n_chunks % NBUF == 0 and NBUF > K >= 1
    mesh = plsc.VectorSubcoreMesh(core_axis_name="c", subcore_axis_name="s")

    @functools.partial(
        pl.kernel,
        out_type=jax.ShapeDtypeStruct((2 * B, 16), jnp.float32),
        mesh=mesh,
        scratch_types=[
            pltpu.VMEM((NBUF, C), jnp.int32),
            pltpu.VMEM((NBUF, 2 * C), jnp.int32),
            pltpu.VMEM((NBUF, 2 * C, 16), jnp.float32),
            pltpu.SemaphoreType.DMA((NBUF,)),
            pltpu.SemaphoreType.DMA((NBUF,)),
        ],
        compiler_params=pltpu.CompilerParams(use_tc_tiling_on_sc=False,
                                             needs_layout_passes=False),
    )
    def k(x_hbm, table_hbm, out_hbm, idx_v, idx2_v, rows_v, s_g, s_st):
        wid = lax.axis_index("s") * _NC + lax.axis_index("c")
        base = wid * b_per_w

        def gather_copy(b):
            return pltpu.make_async_copy(
                table_hbm.at[idx2_v.at[b]], rows_v.at[b], s_g.at[b])

        def store_copy(g, b):
            return pltpu.make_async_copy(
                rows_v.at[b],
                out_hbm.at[pl.ds(2 * (base + g * C), 2 * C)], s_st.at[b])

        def expand_idx(j):
            lanes2 = 2 * lax.iota(jnp.int32, _L)
            src = idx_v.at[j]
            dst = idx2_v.at[j]

            def body(t, carry):
                v = src[pl.ds(t * _L, _L)]
                v2 = v * 2
                pos = 2 * _L * t + lanes2
                plsc.store_scatter(dst, [pos], v2)
                plsc.store_scatter(dst, [pos + 1], v2 + 1)
                return carry

            lax.fori_loop(0, C // _L, body, 0)

        def outer(g2, carry):
            for j in range(NBUF):
                g = g2 * NBUF + j

                @pl.when(g >= NBUF)
                def _():
                    store_copy(g - NBUF, j).wait()

                pltpu.sync_copy(x_hbm.at[pl.ds(base + g * C, C)],
                                idx_v.at[j])
                expand_idx(j)
                gather_copy(j).start()

                jd = (j - K) % NBUF

                @pl.when(g >= K)
                def _():
                    gather_copy(jd).wait()
                    store_copy(g - K, jd).start()

            return carry

        lax.fori_loop(0, n_chunks // NBUF, outer, 0)

        for c in range(n_chunks - K, n_chunks):
            b = c % NBUF
            gather_copy(b).wait()
            store_copy(c, b).start()
        for c in range(n_chunks - NBUF, n_chunks):
            store_copy(c, c % NBUF).wait()

    return k(x_flat, table2)


def kernel(x, table):
    B = x.shape[0] * x.shape[1]
    D = table.shape[1]
    table2 = table.reshape(table.shape[0] * 2, D // 2)
    out2 = _emb_gather(x.reshape(B).astype(jnp.int32), table2, B,
                       1024, 2, 1)
    return out2.reshape(x.shape[0], x.shape[1], D)
